# baseline (device time: 47102 ns/iter reference)
import jax
import jax.numpy as jnp
from jax import lax
from jax.experimental import pallas as pl
from jax.experimental.pallas import tpu as pltpu

N_DEV = 4
E_PER = 2


def kernel(x, router_W, route_idx, expert_W):
    n, d = x.shape
    hdim = expert_W.shape[-1]

    def body(x_ref, rw_ref, idx_ref, w_ref, out_ref, comm_ref, send_sems, recv_sems):
        my_pos = lax.axis_index("i")
        left = lax.rem(my_pos - 1 + N_DEV, N_DEV)
        right = lax.rem(my_pos + 1, N_DEV)

        barrier_sem = pltpu.get_barrier_semaphore()
        for nbr in (left, right):
            pl.semaphore_signal(
                barrier_sem, inc=1,
                device_id=(nbr,), device_id_type=pl.DeviceIdType.MESH,
            )
        pl.semaphore_wait(barrier_sem, 2)

        idx = idx_ref[:, :]
        xv = x_ref[:, :]
        partial = jnp.zeros((n, hdim), jnp.float32)
        for k in range(E_PER):
            e = my_pos * E_PER + k
            m = (idx == e).astype(jnp.float32)
            partial = partial + jnp.dot(
                xv * m, w_ref[k], preferred_element_type=jnp.float32
            )
        comm_ref[0, :, :] = partial
        out_ref[:, :] = partial

        for hop in range(N_DEV - 1):
            rdma = pltpu.make_async_remote_copy(
                src_ref=comm_ref.at[hop],
                dst_ref=comm_ref.at[hop + 1],
                send_sem=send_sems.at[hop],
                recv_sem=recv_sems.at[hop],
                device_id=(right,),
                device_id_type=pl.DeviceIdType.MESH,
            )
            rdma.start()
            rdma.wait()
            out_ref[:, :] = out_ref[:, :] + comm_ref[hop + 1, :, :]

    return pl.pallas_call(
        body,
        out_shape=jax.ShapeDtypeStruct((n, hdim), jnp.float32),
        in_specs=[pl.BlockSpec(memory_space=pltpu.VMEM)] * 4,
        out_specs=pl.BlockSpec(memory_space=pltpu.VMEM),
        scratch_shapes=[
            pltpu.VMEM((N_DEV, n, hdim), jnp.float32),
            pltpu.SemaphoreType.DMA((N_DEV - 1,)),
            pltpu.SemaphoreType.DMA((N_DEV - 1,)),
        ],
        compiler_params=pltpu.CompilerParams(collective_id=0),
    )(x, router_W, route_idx, expert_W)


# device time: 29408 ns/iter; 1.6017x vs baseline; 1.6017x over previous
import jax
import jax.numpy as jnp
from jax import lax
from jax.experimental import pallas as pl
from jax.experimental.pallas import tpu as pltpu

N_DEV = 4
E_PER = 2


def kernel(x, router_W, route_idx, expert_W):
    n, d = x.shape
    hdim = expert_W.shape[-1]
    half, quart = n // 2, n // 4

    def body(x_ref, rw_ref, idx_ref, w_ref, out_ref,
             rbufA, rbufB, send_sems, recv_sems):
        my_pos = lax.axis_index("i")
        pA = my_pos ^ 1
        pB = 3 - my_pos

        kA_off = jnp.where((my_pos == 1) | (my_pos == 2), half, 0)
        notA_off = half - kA_off
        q_off = kA_off + jnp.where(my_pos <= 1, 0, quart)
        oq_off = 2 * kA_off + quart - q_off

        barrier_sem = pltpu.get_barrier_semaphore()
        for nbr in (pA, pB):
            pl.semaphore_signal(
                barrier_sem, inc=1,
                device_id=(nbr,), device_id_type=pl.DeviceIdType.MESH,
            )
        pl.semaphore_wait(barrier_sem, 2)

        idx = idx_ref[:, :]
        xv = x_ref[:, :]
        partial = jnp.zeros((n, hdim), jnp.float32)
        for k in range(E_PER):
            e = my_pos * E_PER + k
            m = (idx == e).astype(jnp.float32)
            partial = partial + jnp.dot(
                xv * m, w_ref[k], preferred_element_type=jnp.float32
            )
        out_ref[:, :] = partial

        rdmaA = pltpu.make_async_remote_copy(
            src_ref=out_ref.at[pl.ds(notA_off, half), :],
            dst_ref=rbufA,
            send_sem=send_sems.at[0], recv_sem=recv_sems.at[0],
            device_id=(pA,), device_id_type=pl.DeviceIdType.MESH,
        )
        rdmaA.start()
        rdmaA.wait()
        out_ref[pl.ds(kA_off, half), :] = (
            out_ref[pl.ds(kA_off, half), :] + rbufA[:, :]
        )

        rdmaB = pltpu.make_async_remote_copy(
            src_ref=out_ref.at[pl.ds(oq_off, quart), :],
            dst_ref=rbufB,
            send_sem=send_sems.at[1], recv_sem=recv_sems.at[1],
            device_id=(pB,), device_id_type=pl.DeviceIdType.MESH,
        )
        rdmaB.start()
        rdmaB.wait()
        out_ref[pl.ds(q_off, quart), :] = (
            out_ref[pl.ds(q_off, quart), :] + rbufB[:, :]
        )

        rdmaC = pltpu.make_async_remote_copy(
            src_ref=out_ref.at[pl.ds(q_off, quart), :],
            dst_ref=out_ref.at[pl.ds(q_off, quart), :],
            send_sem=send_sems.at[2], recv_sem=recv_sems.at[2],
            device_id=(pB,), device_id_type=pl.DeviceIdType.MESH,
        )
        rdmaC.start()
        rdmaC.wait()

        rdmaD = pltpu.make_async_remote_copy(
            src_ref=out_ref.at[pl.ds(kA_off, half), :],
            dst_ref=out_ref.at[pl.ds(kA_off, half), :],
            send_sem=send_sems.at[3], recv_sem=recv_sems.at[3],
            device_id=(pA,), device_id_type=pl.DeviceIdType.MESH,
        )
        rdmaD.start()
        rdmaD.wait()

    return pl.pallas_call(
        body,
        out_shape=jax.ShapeDtypeStruct((n, hdim), jnp.float32),
        in_specs=[pl.BlockSpec(memory_space=pltpu.VMEM)] * 4,
        out_specs=pl.BlockSpec(memory_space=pltpu.VMEM),
        scratch_shapes=[
            pltpu.VMEM((half, hdim), jnp.float32),
            pltpu.VMEM((quart, hdim), jnp.float32),
            pltpu.SemaphoreType.DMA((4,)),
            pltpu.SemaphoreType.DMA((4,)),
        ],
        compiler_params=pltpu.CompilerParams(collective_id=0),
    )(x, router_W, route_idx, expert_W)


# device time: 22985 ns/iter; 2.0492x vs baseline; 1.2794x over previous
import jax
import jax.numpy as jnp
from jax import lax
from jax.experimental import pallas as pl
from jax.experimental.pallas import tpu as pltpu

N_DEV = 4
E_PER = 2


def kernel(x, router_W, route_idx, expert_W):
    n, d = x.shape
    hdim = expert_W.shape[-1]
    q = n // N_DEV

    def body(x_ref, rw_ref, idx_ref, w_ref, out_ref,
             rsbuf, rs_send, rs_recv, ag_send, ag_recv):
        my_pos = lax.axis_index("i")
        peers = [my_pos ^ 2, my_pos ^ 1, 3 - my_pos]

        barrier_sem = pltpu.get_barrier_semaphore()
        for nbr in peers:
            pl.semaphore_signal(
                barrier_sem, inc=1,
                device_id=(nbr,), device_id_type=pl.DeviceIdType.MESH,
            )
        pl.semaphore_wait(barrier_sem, 3)

        def quarter_partial(row_off):
            xq = x_ref[pl.ds(row_off, q), :]
            iq = idx_ref[pl.ds(row_off, q), :]
            acc = jnp.zeros((q, hdim), jnp.float32)
            for k in range(E_PER):
                e = my_pos * E_PER + k
                m = (iq == e).astype(jnp.float32)
                acc = acc + jnp.dot(
                    xq * m, w_ref[k], preferred_element_type=jnp.float32
                )
            return acc

        rs_rdmas = []
        for j in peers:
            slot = lax.rem(my_pos - j - 1 + N_DEV, N_DEV)
            out_ref[pl.ds(j * q, q), :] = quarter_partial(j * q)
            rdma = pltpu.make_async_remote_copy(
                src_ref=out_ref.at[pl.ds(j * q, q), :],
                dst_ref=rsbuf.at[slot],
                send_sem=rs_send.at[slot],
                recv_sem=rs_recv.at[slot],
                device_id=(j,), device_id_type=pl.DeviceIdType.MESH,
            )
            rdma.start()
            rs_rdmas.append(rdma)

        own = quarter_partial(my_pos * q)

        for rdma in rs_rdmas:
            rdma.wait_recv()
        own = own + rsbuf[0] + rsbuf[1] + rsbuf[2]
        out_ref[pl.ds(my_pos * q, q), :] = own

        ag_rdmas = []
        for j in peers:
            slot = lax.rem(my_pos - j - 1 + N_DEV, N_DEV)
            rdma = pltpu.make_async_remote_copy(
                src_ref=out_ref.at[pl.ds(my_pos * q, q), :],
                dst_ref=out_ref.at[pl.ds(my_pos * q, q), :],
                send_sem=ag_send.at[slot],
                recv_sem=ag_recv.at[slot],
                device_id=(j,), device_id_type=pl.DeviceIdType.MESH,
            )
            rdma.start()
            ag_rdmas.append(rdma)

        for rdma in rs_rdmas:
            rdma.wait_send()
        for rdma in ag_rdmas:
            rdma.wait()

    return pl.pallas_call(
        body,
        out_shape=jax.ShapeDtypeStruct((n, hdim), jnp.float32),
        in_specs=[pl.BlockSpec(memory_space=pltpu.VMEM)] * 4,
        out_specs=pl.BlockSpec(memory_space=pltpu.VMEM),
        scratch_shapes=[
            pltpu.VMEM((3, q, hdim), jnp.float32),
            pltpu.SemaphoreType.DMA((3,)),
            pltpu.SemaphoreType.DMA((3,)),
            pltpu.SemaphoreType.DMA((3,)),
            pltpu.SemaphoreType.DMA((3,)),
        ],
        compiler_params=pltpu.CompilerParams(collective_id=0),
    )(x, router_W, route_idx, expert_W)


# device time: 20966 ns/iter; 2.2466x vs baseline; 1.0963x over previous
import jax
import jax.numpy as jnp
from jax import lax
from jax.experimental import pallas as pl
from jax.experimental.pallas import tpu as pltpu

N_DEV = 4
E_PER = 2
N_SUB = 2


def kernel(x, router_W, route_idx, expert_W):
    n, d = x.shape
    hdim = expert_W.shape[-1]
    q = n // N_DEV
    sub = q // N_SUB

    def body(x_ref, rw_ref, idx_ref, w_ref, out_ref,
             rsbuf, rs_send, rs_recv, ag_send, ag_recv):
        my_pos = lax.axis_index("i")
        peers = [my_pos ^ 2, my_pos ^ 1, 3 - my_pos]

        barrier_sem = pltpu.get_barrier_semaphore()
        for nbr in peers:
            pl.semaphore_signal(
                barrier_sem, inc=1,
                device_id=(nbr,), device_id_type=pl.DeviceIdType.MESH,
            )
        pl.semaphore_wait(barrier_sem, 3)

        def quarter_partial(row_off):
            xq = x_ref[pl.ds(row_off, q), :]
            iq = idx_ref[pl.ds(row_off, q), :]
            acc = jnp.zeros((q, hdim), jnp.float32)
            for k in range(E_PER):
                e = my_pos * E_PER + k
                m = (iq == e).astype(jnp.float32)
                acc = acc + jnp.dot(
                    xq * m, w_ref[k], preferred_element_type=jnp.float32
                )
            return acc

        def rs_rdma(j, s):
            slot = lax.rem(my_pos - j - 1 + N_DEV, N_DEV) * N_SUB + s
            return pltpu.make_async_remote_copy(
                src_ref=out_ref.at[pl.ds(j * q + s * sub, sub), :],
                dst_ref=rsbuf.at[slot],
                send_sem=rs_send.at[slot],
                recv_sem=rs_recv.at[slot],
                device_id=(j,), device_id_type=pl.DeviceIdType.MESH,
            )

        def ag_rdma(j, s):
            slot = lax.rem(my_pos - j - 1 + N_DEV, N_DEV) * N_SUB + s
            return pltpu.make_async_remote_copy(
                src_ref=out_ref.at[pl.ds(my_pos * q + s * sub, sub), :],
                dst_ref=out_ref.at[pl.ds(my_pos * q + s * sub, sub), :],
                send_sem=ag_send.at[slot],
                recv_sem=ag_recv.at[slot],
                device_id=(j,), device_id_type=pl.DeviceIdType.MESH,
            )

        rs0, rs1 = [], []
        for j in peers:
            out_ref[pl.ds(j * q, q), :] = quarter_partial(j * q)
            r = rs_rdma(j, 0)
            r.start()
            rs0.append(r)
        for j in peers:
            r = rs_rdma(j, 1)
            r.start()
            rs1.append(r)
        own = quarter_partial(my_pos * q)
        out_ref[pl.ds(my_pos * q, q), :] = own

        ag_rdmas = []
        for s, rs_list in ((0, rs0), (1, rs1)):
            for r in rs_list:
                r.wait_recv()
            off = my_pos * q + s * sub
            out_ref[pl.ds(off, sub), :] = (
                out_ref[pl.ds(off, sub), :]
                + rsbuf[0 * N_SUB + s] + rsbuf[1 * N_SUB + s]
                + rsbuf[2 * N_SUB + s]
            )
            for j in peers:
                r = ag_rdma(j, s)
                r.start()
                ag_rdmas.append(r)

        for r in rs0 + rs1:
            r.wait_send()
        for r in ag_rdmas:
            r.wait()

    return pl.pallas_call(
        body,
        out_shape=jax.ShapeDtypeStruct((n, hdim), jnp.float32),
        in_specs=[pl.BlockSpec(memory_space=pltpu.VMEM)] * 4,
        out_specs=pl.BlockSpec(memory_space=pltpu.VMEM),
        scratch_shapes=[
            pltpu.VMEM((3 * N_SUB, sub, hdim), jnp.float32),
            pltpu.SemaphoreType.DMA((3 * N_SUB,)),
            pltpu.SemaphoreType.DMA((3 * N_SUB,)),
            pltpu.SemaphoreType.DMA((3 * N_SUB,)),
            pltpu.SemaphoreType.DMA((3 * N_SUB,)),
        ],
        compiler_params=pltpu.CompilerParams(collective_id=0),
    )(x, router_W, route_idx, expert_W)


# device time: 20031 ns/iter; 2.3515x vs baseline; 1.0467x over previous
import jax
import jax.numpy as jnp
from jax import lax
from jax.experimental import pallas as pl
from jax.experimental.pallas import tpu as pltpu

N_DEV = 4
E_PER = 2
N_SUB = 2


def kernel(x, router_W, route_idx, expert_W):
    n, d = x.shape
    hdim = expert_W.shape[-1]
    q = n // N_DEV
    sub = q // N_SUB

    def body(x_ref, rw_ref, idx_ref, w_ref, out_ref,
             rsbuf, rs_send, rs_recv, ag_send, ag_recv):
        my_pos = lax.axis_index("i")
        peers = [my_pos ^ 2, my_pos ^ 1, 3 - my_pos]

        barrier_sem = pltpu.get_barrier_semaphore()
        for nbr in peers:
            pl.semaphore_signal(
                barrier_sem, inc=1,
                device_id=(nbr,), device_id_type=pl.DeviceIdType.MESH,
            )

        wcat = w_ref[:, :, :].astype(jnp.bfloat16).reshape(E_PER * d, hdim)

        def quarter_partial(row_off):
            xq = x_ref[pl.ds(row_off, q), :].astype(jnp.bfloat16)
            iq = idx_ref[pl.ds(row_off, q), :]
            masked = [
                xq * (iq == my_pos * E_PER + k).astype(jnp.bfloat16)
                for k in range(E_PER)
            ]
            return jnp.dot(
                jnp.concatenate(masked, axis=1), wcat,
                preferred_element_type=jnp.float32,
            )

        def rs_rdma(j, s):
            slot = lax.rem(my_pos - j - 1 + N_DEV, N_DEV) * N_SUB + s
            return pltpu.make_async_remote_copy(
                src_ref=out_ref.at[pl.ds(j * q + s * sub, sub), :],
                dst_ref=rsbuf.at[slot],
                send_sem=rs_send.at[slot],
                recv_sem=rs_recv.at[slot],
                device_id=(j,), device_id_type=pl.DeviceIdType.MESH,
            )

        def ag_rdma(j, s):
            slot = lax.rem(my_pos - j - 1 + N_DEV, N_DEV) * N_SUB + s
            return pltpu.make_async_remote_copy(
                src_ref=out_ref.at[pl.ds(my_pos * q + s * sub, sub), :],
                dst_ref=out_ref.at[pl.ds(my_pos * q + s * sub, sub), :],
                send_sem=ag_send.at[slot],
                recv_sem=ag_recv.at[slot],
                device_id=(j,), device_id_type=pl.DeviceIdType.MESH,
            )

        rs0, rs1 = [], []
        for pi, j in enumerate(peers):
            out_ref[pl.ds(j * q, q), :] = quarter_partial(j * q)
            if pi == 0:
                pl.semaphore_wait(barrier_sem, 3)
            r = rs_rdma(j, 0)
            r.start()
            rs0.append(r)
        for j in peers:
            r = rs_rdma(j, 1)
            r.start()
            rs1.append(r)
        own = quarter_partial(my_pos * q)
        out_ref[pl.ds(my_pos * q, q), :] = own

        ag_rdmas = []
        for s, rs_list in ((0, rs0), (1, rs1)):
            for r in rs_list:
                r.wait_recv()
            off = my_pos * q + s * sub
            out_ref[pl.ds(off, sub), :] = (
                out_ref[pl.ds(off, sub), :]
                + rsbuf[0 * N_SUB + s] + rsbuf[1 * N_SUB + s]
                + rsbuf[2 * N_SUB + s]
            )
            for j in peers:
                r = ag_rdma(j, s)
                r.start()
                ag_rdmas.append(r)

        for r in rs0 + rs1:
            r.wait_send()
        for r in ag_rdmas:
            r.wait()

    return pl.pallas_call(
        body,
        out_shape=jax.ShapeDtypeStruct((n, hdim), jnp.float32),
        in_specs=[pl.BlockSpec(memory_space=pltpu.VMEM)] * 4,
        out_specs=pl.BlockSpec(memory_space=pltpu.VMEM),
        scratch_shapes=[
            pltpu.VMEM((3 * N_SUB, sub, hdim), jnp.float32),
            pltpu.SemaphoreType.DMA((3 * N_SUB,)),
            pltpu.SemaphoreType.DMA((3 * N_SUB,)),
            pltpu.SemaphoreType.DMA((3 * N_SUB,)),
            pltpu.SemaphoreType.DMA((3 * N_SUB,)),
        ],
        compiler_params=pltpu.CompilerParams(collective_id=0),
    )(x, router_W, route_idx, expert_W)


# device time: 18579 ns/iter; 2.5352x vs baseline; 1.0782x over previous
import jax
import jax.numpy as jnp
from jax import lax
from jax.experimental import pallas as pl
from jax.experimental.pallas import tpu as pltpu

N_DEV = 4
E_PER = 2
N_SUB = 2


def kernel(x, router_W, route_idx, expert_W):
    n, d = x.shape
    hdim = expert_W.shape[-1]
    q = n // N_DEV
    sub = q // N_SUB

    def body(x_ref, rw_ref, idx_ref, w_ref, out_ref,
             txbuf, rsbuf, rs_send, rs_recv, ag_send, ag_recv):
        my_pos = lax.axis_index("i")
        peers = [my_pos ^ 2, my_pos ^ 1, 3 - my_pos]

        barrier_sem = pltpu.get_barrier_semaphore()
        for nbr in peers:
            pl.semaphore_signal(
                barrier_sem, inc=1,
                device_id=(nbr,), device_id_type=pl.DeviceIdType.MESH,
            )

        wcat = w_ref[:, :, :].astype(jnp.bfloat16).reshape(E_PER * d, hdim)

        def quarter_partial(row_off):
            xq = x_ref[pl.ds(row_off, q), :].astype(jnp.bfloat16)
            iq = idx_ref[pl.ds(row_off, q), :]
            masked = [
                xq * (iq == my_pos * E_PER + k).astype(jnp.bfloat16)
                for k in range(E_PER)
            ]
            return jnp.dot(
                jnp.concatenate(masked, axis=1), wcat,
                preferred_element_type=jnp.float32,
            )

        def rs_rdma(j, s):
            slot = lax.rem(my_pos - j - 1 + N_DEV, N_DEV) * N_SUB + s
            return pltpu.make_async_remote_copy(
                src_ref=txbuf.at[pl.ds(j * q + s * sub, sub), :],
                dst_ref=rsbuf.at[slot],
                send_sem=rs_send.at[slot],
                recv_sem=rs_recv.at[slot],
                device_id=(j,), device_id_type=pl.DeviceIdType.MESH,
            )

        def ag_rdma(j, s):
            slot = lax.rem(my_pos - j - 1 + N_DEV, N_DEV) * N_SUB + s
            return pltpu.make_async_remote_copy(
                src_ref=out_ref.at[pl.ds(my_pos * q + s * sub, sub), :],
                dst_ref=out_ref.at[pl.ds(my_pos * q + s * sub, sub), :],
                send_sem=ag_send.at[slot],
                recv_sem=ag_recv.at[slot],
                device_id=(j,), device_id_type=pl.DeviceIdType.MESH,
            )

        rs0, rs1 = [], []
        for pi, j in enumerate(peers):
            txbuf[pl.ds(j * q, q), :] = quarter_partial(j * q).astype(jnp.bfloat16)
            if pi == 0:
                pl.semaphore_wait(barrier_sem, 3)
            r = rs_rdma(j, 0)
            r.start()
            rs0.append(r)
        for j in peers:
            r = rs_rdma(j, 1)
            r.start()
            rs1.append(r)
        own = quarter_partial(my_pos * q)
        out_ref[pl.ds(my_pos * q, q), :] = own

        ag_rdmas = []
        for s, rs_list in ((0, rs0), (1, rs1)):
            for r in rs_list:
                r.wait_recv()
            off = my_pos * q + s * sub
            out_ref[pl.ds(off, sub), :] = (
                out_ref[pl.ds(off, sub), :]
                + rsbuf[0 * N_SUB + s].astype(jnp.float32)
                + rsbuf[1 * N_SUB + s].astype(jnp.float32)
                + rsbuf[2 * N_SUB + s].astype(jnp.float32)
            )
            for j in peers:
                r = ag_rdma(j, s)
                r.start()
                ag_rdmas.append(r)

        for r in rs0 + rs1:
            r.wait_send()
        for r in ag_rdmas:
            r.wait()

    return pl.pallas_call(
        body,
        out_shape=jax.ShapeDtypeStruct((n, hdim), jnp.float32),
        in_specs=[pl.BlockSpec(memory_space=pltpu.VMEM)] * 4,
        out_specs=pl.BlockSpec(memory_space=pltpu.VMEM),
        scratch_shapes=[
            pltpu.VMEM((n, hdim), jnp.bfloat16),
            pltpu.VMEM((3 * N_SUB, sub, hdim), jnp.bfloat16),
            pltpu.SemaphoreType.DMA((3 * N_SUB,)),
            pltpu.SemaphoreType.DMA((3 * N_SUB,)),
            pltpu.SemaphoreType.DMA((3 * N_SUB,)),
            pltpu.SemaphoreType.DMA((3 * N_SUB,)),
        ],
        compiler_params=pltpu.CompilerParams(collective_id=0),
    )(x, router_W, route_idx, expert_W)


# device time: 15779 ns/iter; 2.9851x vs baseline; 1.1775x over previous
import jax
import jax.numpy as jnp
from jax import lax
from jax.experimental import pallas as pl
from jax.experimental.pallas import tpu as pltpu

N_DEV = 4
E_PER = 2
N_SUB = 2


def kernel(x, router_W, route_idx, expert_W):
    n, d = x.shape
    hdim = expert_W.shape[-1]
    q = n // N_DEV
    sub = q // N_SUB

    def body(x_ref, rw_ref, idx_ref, w_ref, out_ref,
             txbuf, rsbuf, rs_send, rs_recv, ag_send, ag_recv):
        my_pos = lax.axis_index("i")
        peers = [my_pos ^ 2, my_pos ^ 1, 3 - my_pos]

        barrier_sem = pltpu.get_barrier_semaphore()
        for nbr in peers:
            pl.semaphore_signal(
                barrier_sem, inc=1,
                device_id=(nbr,), device_id_type=pl.DeviceIdType.MESH,
            )

        wcat = w_ref[:, :, :].astype(jnp.bfloat16).reshape(E_PER * d, hdim)

        def quarter_partial(row_off):
            xq = x_ref[pl.ds(row_off, q), :].astype(jnp.bfloat16)
            iq = idx_ref[pl.ds(row_off, q), :]
            masked = [
                xq * (iq == my_pos * E_PER + k).astype(jnp.bfloat16)
                for k in range(E_PER)
            ]
            return jnp.dot(
                jnp.concatenate(masked, axis=1), wcat,
                preferred_element_type=jnp.float32,
            )

        def rs_rdma(j, s):
            slot = lax.rem(my_pos - j - 1 + N_DEV, N_DEV) * N_SUB + s
            return pltpu.make_async_remote_copy(
                src_ref=txbuf.at[pl.ds(j * q + s * sub, sub), :],
                dst_ref=rsbuf.at[slot],
                send_sem=rs_send.at[slot],
                recv_sem=rs_recv.at[slot],
                device_id=(j,), device_id_type=pl.DeviceIdType.MESH,
            )

        def ag_rdma(j, s):
            slot = lax.rem(my_pos - j - 1 + N_DEV, N_DEV) * N_SUB + s
            return pltpu.make_async_remote_copy(
                src_ref=txbuf.at[pl.ds(my_pos * q + s * sub, sub), :],
                dst_ref=txbuf.at[pl.ds(my_pos * q + s * sub, sub), :],
                send_sem=ag_send.at[slot],
                recv_sem=ag_recv.at[slot],
                device_id=(j,), device_id_type=pl.DeviceIdType.MESH,
            )

        rs0, rs1 = [], []
        for pi, j in enumerate(peers):
            txbuf[pl.ds(j * q, q), :] = quarter_partial(j * q).astype(jnp.bfloat16)
            if pi == 0:
                pl.semaphore_wait(barrier_sem, 3)
            r = rs_rdma(j, 0)
            r.start()
            rs0.append(r)
        for j in peers:
            r = rs_rdma(j, 1)
            r.start()
            rs1.append(r)
        own = quarter_partial(my_pos * q)
        out_ref[pl.ds(my_pos * q, q), :] = own

        ag_rdmas = []
        for s, rs_list in ((0, rs0), (1, rs1)):
            for r in rs_list:
                r.wait_recv()
            off = my_pos * q + s * sub
            red = (
                out_ref[pl.ds(off, sub), :]
                + rsbuf[0 * N_SUB + s].astype(jnp.float32)
                + rsbuf[1 * N_SUB + s].astype(jnp.float32)
                + rsbuf[2 * N_SUB + s].astype(jnp.float32)
            )
            out_ref[pl.ds(off, sub), :] = red
            txbuf[pl.ds(off, sub), :] = red.astype(jnp.bfloat16)
            for j in peers:
                r = ag_rdma(j, s)
                r.start()
                t = lax.rem(my_pos - j - 1 + N_DEV, N_DEV)
                sender = lax.rem(my_pos + t + 1, N_DEV)
                ag_rdmas.append((r, sender, s))

        for r in rs0 + rs1:
            r.wait_send()
        for r, sender, s in ag_rdmas:
            r.wait_recv()
            off = sender * q + s * sub
            out_ref[pl.ds(off, sub), :] = (
                txbuf[pl.ds(off, sub), :].astype(jnp.float32)
            )
        for r, _, _ in ag_rdmas:
            r.wait_send()

    return pl.pallas_call(
        body,
        out_shape=jax.ShapeDtypeStruct((n, hdim), jnp.float32),
        in_specs=[pl.BlockSpec(memory_space=pltpu.VMEM)] * 4,
        out_specs=pl.BlockSpec(memory_space=pltpu.VMEM),
        scratch_shapes=[
            pltpu.VMEM((n, hdim), jnp.bfloat16),
            pltpu.VMEM((3 * N_SUB, sub, hdim), jnp.bfloat16),
            pltpu.SemaphoreType.DMA((3 * N_SUB,)),
            pltpu.SemaphoreType.DMA((3 * N_SUB,)),
            pltpu.SemaphoreType.DMA((3 * N_SUB,)),
            pltpu.SemaphoreType.DMA((3 * N_SUB,)),
        ],
        compiler_params=pltpu.CompilerParams(collective_id=0),
    )(x, router_W, route_idx, expert_W)
